# baseline (speedup 1.0000x reference)
import jax
import jax.numpy as jnp
from jax.experimental import pallas as pl
from jax.experimental.pallas import tpu as pltpu

_K = 1_000_000
_D = 64
_Q = 16
_GRP = 8
_W = _GRP * _D
_KR = _K // _GRP
_CHR = 1000
_NCH = _KR // _CHR
_NBUF = 16


def _nn_kernel(b1_ref, qnt_ref, hbm_ref, dist_ref, idx_ref, buf_ref, sem_ref):
    j = pl.program_id(0)

    def _copy(ch, b):
        return pltpu.make_async_copy(
            hbm_ref.at[pl.ds(ch * _CHR, _CHR), :],
            buf_ref.at[b],
            sem_ref.at[b])

    @pl.when(j == 0)
    def _init():
        dist_ref[...] = jnp.full_like(dist_ref, jnp.inf)
        idx_ref[...] = jnp.zeros_like(idx_ref)
        for t in range(_NBUF - 1):
            _copy(t, t).start()

    nxt = j + _NBUF - 1

    @pl.when(nxt < _NCH)
    def _issue():
        _copy(nxt, nxt % _NBUF).start()

    b = j % _NBUF
    _copy(j, b).wait()
    c = buf_ref[b]

    b1 = b1_ref[...]
    qnt = qnt_ref[...]
    row2 = jax.lax.broadcasted_iota(jnp.int32, (_W, _GRP * _Q), 0)
    col2 = jax.lax.broadcasted_iota(jnp.int32, (_W, _GRP * _Q), 1)
    ones_b = (row2 // _D == col2 // _Q).astype(jnp.float32)
    lane = jax.lax.broadcasted_iota(jnp.int32, (1, _GRP * _Q), 1)
    g_of_lane = lane // _Q

    m1 = jnp.dot(c, b1, preferred_element_type=jnp.float32)
    m2 = jnp.dot(c * c, ones_b, preferred_element_type=jnp.float32)
    d2 = (m1 + m2) + qnt
    lmin = jnp.min(d2, axis=0, keepdims=True)
    lrow = jnp.argmin(d2, axis=0).astype(jnp.int32)[None, :]
    gidx = _GRP * (j * _CHR + lrow) + g_of_lane
    better = lmin < dist_ref[...]
    dist_ref[...] = jnp.where(better, lmin, dist_ref[...])
    idx_ref[...] = jnp.where(better, gidx, idx_ref[...])

    @pl.when(j == _NCH - 1)
    def _finish():
        bd = dist_ref[0:1, 0:_Q]
        bi = idx_ref[0:1, 0:_Q]
        for g in range(1, _GRP):
            vd = dist_ref[0:1, g * _Q:(g + 1) * _Q]
            vi = idx_ref[0:1, g * _Q:(g + 1) * _Q]
            upd = vd < bd
            bd = jnp.where(upd, vd, bd)
            bi = jnp.where(upd, vi, bi)
        dist_ref[0:1, 0:_Q] = jnp.sqrt(jnp.maximum(bd, 0.0))
        idx_ref[0:1, 0:_Q] = bi


def kernel(face_embedding, centroids):
    qt = face_embedding.T
    b1 = -2.0 * jnp.kron(jnp.eye(_GRP, dtype=jnp.float32), qt)
    qn = jnp.sum(face_embedding * face_embedding, axis=1)
    qnt = jnp.tile(qn, _GRP)[None, :]
    cpacked = centroids.reshape(_KR, _W)
    dist, idx = pl.pallas_call(
        _nn_kernel,
        grid=(_NCH,),
        in_specs=[
            pl.BlockSpec((_W, _GRP * _Q), lambda i: (0, 0)),
            pl.BlockSpec((1, _GRP * _Q), lambda i: (0, 0)),
            pl.BlockSpec(memory_space=pltpu.MemorySpace.HBM),
        ],
        out_specs=[
            pl.BlockSpec((1, _GRP * _Q), lambda i: (0, 0)),
            pl.BlockSpec((1, _GRP * _Q), lambda i: (0, 0)),
        ],
        out_shape=[
            jax.ShapeDtypeStruct((1, _GRP * _Q), jnp.float32),
            jax.ShapeDtypeStruct((1, _GRP * _Q), jnp.int32),
        ],
        scratch_shapes=[
            pltpu.VMEM((_NBUF, _CHR, _W), jnp.float32),
            pltpu.SemaphoreType.DMA((_NBUF,)),
        ],
    )(b1, qnt, cpacked)
    return dist[0, :_Q], idx[0, :_Q]


# speedup vs baseline: 1.0129x; 1.0129x over previous
import jax
import jax.numpy as jnp
from jax.experimental import pallas as pl
from jax.experimental.pallas import tpu as pltpu

_K = 1_000_000
_D = 64
_Q = 16
_GRP = 2
_W = _GRP * _D
_KR = _K // _GRP
_CHR = 4000
_NCH = _KR // _CHR
_NBUF = 8


def _nn_kernel(b1_ref, qnt_ref, hbm_ref, dist_ref, idx_ref, buf_ref, sem_ref):
    j = pl.program_id(0)

    def _copy(ch, b):
        return pltpu.make_async_copy(
            hbm_ref.at[pl.ds(ch * _CHR, _CHR), :],
            buf_ref.at[b],
            sem_ref.at[b])

    @pl.when(j == 0)
    def _init():
        dist_ref[...] = jnp.full_like(dist_ref, jnp.inf)
        idx_ref[...] = jnp.zeros_like(idx_ref)
        for t in range(_NBUF - 1):
            _copy(t, t).start()

    nxt = j + _NBUF - 1

    @pl.when(nxt < _NCH)
    def _issue():
        _copy(nxt, nxt % _NBUF).start()

    b = j % _NBUF
    _copy(j, b).wait()
    c = buf_ref[b]

    b1 = b1_ref[...]
    qnt = qnt_ref[...]
    row2 = jax.lax.broadcasted_iota(jnp.int32, (_W, _GRP * _Q), 0)
    col2 = jax.lax.broadcasted_iota(jnp.int32, (_W, _GRP * _Q), 1)
    ones_b = (row2 // _D == col2 // _Q).astype(jnp.float32)
    lane = jax.lax.broadcasted_iota(jnp.int32, (1, _GRP * _Q), 1)
    g_of_lane = lane // _Q

    m1 = jnp.dot(c, b1, preferred_element_type=jnp.float32)
    m2 = jnp.dot(c * c, ones_b, preferred_element_type=jnp.float32)
    d2 = (m1 + m2) + qnt
    lmin = jnp.min(d2, axis=0, keepdims=True)
    lrow = jnp.argmin(d2, axis=0).astype(jnp.int32)[None, :]
    gidx = _GRP * (j * _CHR + lrow) + g_of_lane
    better = lmin < dist_ref[...]
    dist_ref[...] = jnp.where(better, lmin, dist_ref[...])
    idx_ref[...] = jnp.where(better, gidx, idx_ref[...])

    @pl.when(j == _NCH - 1)
    def _finish():
        bd = dist_ref[0:1, 0:_Q]
        bi = idx_ref[0:1, 0:_Q]
        for g in range(1, _GRP):
            vd = dist_ref[0:1, g * _Q:(g + 1) * _Q]
            vi = idx_ref[0:1, g * _Q:(g + 1) * _Q]
            upd = vd < bd
            bd = jnp.where(upd, vd, bd)
            bi = jnp.where(upd, vi, bi)
        dist_ref[0:1, 0:_Q] = jnp.sqrt(jnp.maximum(bd, 0.0))
        idx_ref[0:1, 0:_Q] = bi


def kernel(face_embedding, centroids):
    qt = face_embedding.T
    b1 = -2.0 * jnp.kron(jnp.eye(_GRP, dtype=jnp.float32), qt)
    qn = jnp.sum(face_embedding * face_embedding, axis=1)
    qnt = jnp.tile(qn, _GRP)[None, :]
    cpacked = centroids.reshape(_KR, _W)
    dist, idx = pl.pallas_call(
        _nn_kernel,
        grid=(_NCH,),
        in_specs=[
            pl.BlockSpec((_W, _GRP * _Q), lambda i: (0, 0)),
            pl.BlockSpec((1, _GRP * _Q), lambda i: (0, 0)),
            pl.BlockSpec(memory_space=pltpu.MemorySpace.HBM),
        ],
        out_specs=[
            pl.BlockSpec((1, _GRP * _Q), lambda i: (0, 0)),
            pl.BlockSpec((1, _GRP * _Q), lambda i: (0, 0)),
        ],
        out_shape=[
            jax.ShapeDtypeStruct((1, _GRP * _Q), jnp.float32),
            jax.ShapeDtypeStruct((1, _GRP * _Q), jnp.int32),
        ],
        scratch_shapes=[
            pltpu.VMEM((_NBUF, _CHR, _W), jnp.float32),
            pltpu.SemaphoreType.DMA((_NBUF,)),
        ],
    )(b1, qnt, cpacked)
    return dist[0, :_Q], idx[0, :_Q]


# speedup vs baseline: 1.3516x; 1.3343x over previous
import jax
import jax.numpy as jnp
from jax.experimental import pallas as pl
from jax.experimental.pallas import tpu as pltpu

_K = 1_000_000
_D = 64
_Q = 16
_P = 5
_BK = 4000
_NCH = _K // _BK
_NST = _NCH // _P
_LA = 2


def _nn_kernel(qt_ref, hbm_ref, dist_ref, idx_ref, buf_ref, sem_ref):
    j = pl.program_id(0)

    def _copy(step, p, slot):
        ch = step * _P + p
        return pltpu.make_async_copy(
            hbm_ref.at[pl.ds(ch * _BK, _BK), :],
            buf_ref.at[slot, p],
            sem_ref.at[slot, p])

    @pl.when(j == 0)
    def _init():
        dist_ref[...] = jnp.full_like(dist_ref, jnp.inf)
        idx_ref[...] = jnp.zeros_like(idx_ref)
        for t in range(_LA):
            for p in range(_P):
                _copy(t, p, t).start()

    nxt = j + _LA

    @pl.when(nxt < _NST)
    def _issue():
        for p in range(_P):
            _copy(nxt, p, nxt % (_LA + 1)).start()

    qt = qt_ref[...]
    qn = jnp.sum(qt * qt, axis=0, keepdims=True)
    slot = j % (_LA + 1)
    for p in range(_P):
        _copy(j, p, slot).wait()
        c = buf_ref[slot, p]
        dots = jnp.dot(c, qt, preferred_element_type=jnp.float32)
        cn = jnp.sum(c * c, axis=1, keepdims=True)
        d2 = (cn + qn) - 2.0 * dots
        lmin = jnp.min(d2, axis=0, keepdims=True)
        lidx = (jnp.argmin(d2, axis=0).astype(jnp.int32)[None, :]
                + (j * _P + p) * _BK)
        better = lmin < dist_ref[...]
        dist_ref[...] = jnp.where(better, lmin, dist_ref[...])
        idx_ref[...] = jnp.where(better, lidx, idx_ref[...])

    @pl.when(j == _NST - 1)
    def _finish():
        dist_ref[...] = jnp.sqrt(jnp.maximum(dist_ref[...], 0.0))


def kernel(face_embedding, centroids):
    qt = face_embedding.T
    dist, idx = pl.pallas_call(
        _nn_kernel,
        grid=(_NST,),
        in_specs=[
            pl.BlockSpec((_D, _Q), lambda i: (0, 0)),
            pl.BlockSpec(memory_space=pltpu.MemorySpace.HBM),
        ],
        out_specs=[
            pl.BlockSpec((1, _Q), lambda i: (0, 0)),
            pl.BlockSpec((1, _Q), lambda i: (0, 0)),
        ],
        out_shape=[
            jax.ShapeDtypeStruct((1, _Q), jnp.float32),
            jax.ShapeDtypeStruct((1, _Q), jnp.int32),
        ],
        scratch_shapes=[
            pltpu.VMEM((_LA + 1, _P, _BK, _D), jnp.float32),
            pltpu.SemaphoreType.DMA((_LA + 1, _P)),
        ],
    )(qt, centroids)
    return dist.reshape(_Q), idx.reshape(_Q)
